# baseline (speedup 1.0000x reference)
import functools

import jax
import jax.numpy as jnp
from jax import lax
from jax.experimental import pallas as pl
from jax.experimental.pallas import tpu as pltpu
from jax.experimental.pallas import tpu_sc as plsc

NC = 2
NS = 16
CHUNK = 128
ZROWS = 64
BLK = 4000


def _elu(v):
    return jnp.where(v > 0, v, jnp.exp(jnp.minimum(v, 0.0)) - 1.0)


def _pad16(s):
    return (s + 15) // 16 * 16


@functools.cache
def _make_seg_sum(n_nodes, e_pad, P, R, passes):
    RA = R + 128
    assert R % 128 == 0
    ra_per = RA // NS
    z_full, z_rem = divmod(ra_per, ZROWS)
    r_per = R // NS
    out_rows = passes * NC * R
    int_min = jnp.int32(-2147483648)

    mesh = plsc.VectorSubcoreMesh(
        core_axis_name="c", subcore_axis_name="s",
        num_cores=NC, num_subcores=NS)

    @functools.partial(
        pl.kernel,
        out_type=jax.ShapeDtypeStruct((out_rows, P), jnp.float32),
        mesh=mesh,
        scratch_types=[
            pltpu.VMEM((CHUNK,), jnp.int32),
            pltpu.VMEM((CHUNK,), jnp.int32),
            pltpu.VMEM((CHUNK, P), jnp.float32),
            pltpu.VMEM((ZROWS, P), jnp.float32),
            pltpu.VMEM((16,), jnp.int32),
            pltpu.VMEM_SHARED((RA, P), jnp.float32),
            pltpu.SemaphoreType.DMA,
        ],
        compiler_params=pltpu.CompilerParams(use_tc_tiling_on_sc=False),
    )
    def seg(vals_hbm, ssrc_hbm, sdst_hbm, bounds_hbm, out_hbm,
            src_v, dst_v, rows_v, zero_v, bnd_v, acc_sh, sem):
        c = lax.axis_index("c")
        s = lax.axis_index("s")
        lanes = lax.iota(jnp.int32, 16)
        zvec = jnp.zeros((16,), jnp.float32)
        for r in range(ZROWS):
            for j in range(P // 16):
                zero_v[r, pl.ds(j * 16, 16)] = zvec

        for p in range(passes):
            zbase = s * ra_per
            def zero_body(i, _):
                pltpu.sync_copy(zero_v, acc_sh.at[pl.ds(zbase + i * ZROWS, ZROWS)])
                return 0
            lax.fori_loop(0, z_full, zero_body, 0)
            if z_rem:
                pltpu.sync_copy(zero_v.at[pl.ds(0, z_rem)],
                                acc_sh.at[pl.ds(zbase + z_full * ZROWS, z_rem)])
            plsc.subcore_barrier()

            pltpu.sync_copy(bounds_hbm.at[p, c], bnd_v)
            bv = bnd_v[...]
            e_lo = bv[0]
            e_hi = bv[1]
            node_lo = (p * NC + c) * R
            nchunks = lax.shift_right_arithmetic(e_hi - e_lo + (CHUNK - 1), 7)
            nloc = lax.shift_right_arithmetic(nchunks - s + (NS - 1), 4)

            def chunk_body(i, _):
                base = pl.multiple_of(e_lo + (s + i * NS) * CHUNK, 8)
                pltpu.sync_copy(ssrc_hbm.at[pl.ds(base, CHUNK)], src_v)
                pltpu.sync_copy(sdst_hbm.at[pl.ds(base, CHUNK)], dst_v)
                for j in range(CHUNK // 16):
                    d = dst_v[pl.ds(j * 16, 16)]
                    loc = d - node_lo
                    ok = (loc >= 0) & (loc < R)
                    dst_v[pl.ds(j * 16, 16)] = jnp.where(ok, loc, R)
                pltpu.async_copy(vals_hbm.at[src_v], rows_v, sem).wait()
                pltpu.sync_copy(rows_v, acc_sh.at[dst_v], add=True)
                return 0
            lax.fori_loop(0, nloc, chunk_body, 0)
            plsc.subcore_barrier()

            out_base = node_lo + s * r_per
            pltpu.sync_copy(acc_sh.at[pl.ds(s * r_per, r_per)],
                            out_hbm.at[pl.ds(out_base, r_per)])
            if p + 1 < passes:
                plsc.subcore_barrier()

    return seg


def _mm_layer(x_parts, b_prev, s_prev, invn, W, s_cur):
    n = invn.shape[0]
    fin, fout = W.shape
    P_cur = _pad16(s_cur)
    grid = (n // BLK,)

    if b_prev is None:
        (pos,) = x_parts

        def body(x_ref, invn_ref, w_ref, norm_ref, rest_ref):
            sup = jnp.dot(x_ref[...], w_ref[...],
                          preferred_element_type=jnp.float32)
            scaled = sup[:, :s_cur] * invn_ref[...]
            norm_ref[...] = jnp.concatenate(
                [scaled, jnp.zeros((BLK, P_cur - s_cur), jnp.float32)], axis=1)
            rest_ref[...] = sup[:, s_cur:]

        in_specs = [
            pl.BlockSpec((BLK, pos.shape[1]), lambda i: (i, 0)),
            pl.BlockSpec((BLK, 1), lambda i: (i, 0)),
            pl.BlockSpec((fin, fout), lambda i: (0, 0)),
        ]
        args = (pos, invn, W)
    else:
        side1, rest_in = x_parts
        r_prev = rest_in.shape[1]

        def body(s1_ref, rest_ref_in, b_ref, invn_ref, w_ref,
                 norm_ref, rest_ref):
            x = jnp.concatenate([s1_ref[:, :s_prev], rest_ref_in[...]],
                                axis=1) + b_ref[...]
            x = _elu(x)
            sup = jnp.dot(x, w_ref[...], preferred_element_type=jnp.float32)
            scaled = sup[:, :s_cur] * invn_ref[...]
            norm_ref[...] = jnp.concatenate(
                [scaled, jnp.zeros((BLK, P_cur - s_cur), jnp.float32)], axis=1)
            rest_ref[...] = sup[:, s_cur:]

        in_specs = [
            pl.BlockSpec((BLK, side1.shape[1]), lambda i: (i, 0)),
            pl.BlockSpec((BLK, r_prev), lambda i: (i, 0)),
            pl.BlockSpec((1, fin), lambda i: (0, 0)),
            pl.BlockSpec((BLK, 1), lambda i: (i, 0)),
            pl.BlockSpec((fin, fout), lambda i: (0, 0)),
        ]
        args = (side1, rest_in, b_prev, invn, W)

    return pl.pallas_call(
        body,
        grid=grid,
        in_specs=in_specs,
        out_specs=[
            pl.BlockSpec((BLK, P_cur), lambda i: (i, 0)),
            pl.BlockSpec((BLK, fout - s_cur), lambda i: (i, 0)),
        ],
        out_shape=[
            jax.ShapeDtypeStruct((n, P_cur), jnp.float32),
            jax.ShapeDtypeStruct((n, fout - s_cur), jnp.float32),
        ],
    )(*args)


def _max_reduce(side1, rest, b, s_prev, n):
    fout = b.shape[0]

    def body(s1_ref, rest_ref, b_ref, out_ref):
        i = pl.program_id(0)
        blk = jnp.concatenate([s1_ref[:, :s_prev], rest_ref[...]], axis=1)
        m = jnp.max(blk, axis=0, keepdims=True)

        @pl.when(i == 0)
        def _():
            out_ref[...] = m

        @pl.when(i > 0)
        def _():
            out_ref[...] = jnp.maximum(out_ref[...], m)

        @pl.when(i == pl.num_programs(0) - 1)
        def _():
            out_ref[...] = _elu(out_ref[...] + b_ref[...])

    out = pl.pallas_call(
        body,
        grid=(n // BLK,),
        in_specs=[
            pl.BlockSpec((BLK, side1.shape[1]), lambda i: (i, 0)),
            pl.BlockSpec((BLK, rest.shape[1]), lambda i: (i, 0)),
            pl.BlockSpec((1, fout), lambda i: (0, 0)),
        ],
        out_specs=pl.BlockSpec((1, fout), lambda i: (0, 0)),
        out_shape=jax.ShapeDtypeStruct((1, fout), jnp.float32),
    )(side1, rest, b.reshape(1, fout))
    return out.reshape(fout)


def _seg_plan(P):
    if P <= 64:
        return 25088, 2
    return 12800, 4


def kernel(positions, adj, params):
    n = positions.shape[0]
    e = adj.shape[1]
    src = adj[0].astype(jnp.int32)
    dst = adj[1].astype(jnp.int32)

    sdst, ssrc = lax.sort_key_val(dst, src)
    rp = jnp.searchsorted(
        sdst, jnp.arange(n + 1, dtype=jnp.int32), side="left").astype(jnp.int32)
    deg = (rp[1:] - rp[:-1]).astype(jnp.float32)
    invn = (1.0 / jnp.maximum(deg, 1.0))[:, None]
    ssrc_pad = jnp.concatenate([ssrc, jnp.zeros((CHUNK,), jnp.int32)])
    sdst_pad = jnp.concatenate([sdst, jnp.full((CHUNK,), n, jnp.int32)])
    e_pad = e + CHUNK

    def mk_bounds(R, passes):
        units = jnp.arange(passes * NC, dtype=jnp.int32) * R
        lo = jnp.minimum(units, n)
        hi = jnp.minimum(units + R, n)
        a = (rp[lo] // 8) * 8
        e_hi = rp[hi]
        bounds = jnp.zeros((passes * NC, 16), jnp.int32)
        bounds = bounds.at[:, 0].set(a).at[:, 1].set(e_hi)
        return bounds.reshape(passes, NC, 16)

    plans = {}
    for (_, fo) in [(None, W.shape[1]) for (W, _) in params]:
        P = _pad16(max(fo // 3, 2))
        if P not in plans:
            plans[P] = _seg_plan(P)
    bounds = {P: mk_bounds(R, passes) for P, (R, passes) in plans.items()}

    x_parts = (positions,)
    b_prev = None
    s_prev = None
    for li, (W, b) in enumerate(params):
        fout = W.shape[1]
        s_cur = max(fout // 3, 2)
        P = _pad16(s_cur)
        R, passes = plans[P]
        normed, rest = _mm_layer(x_parts, b_prev, s_prev, invn, W, s_cur)
        seg = _make_seg_sum(n, e_pad, P, R, passes)
        side1 = seg(normed, ssrc_pad, sdst_pad, bounds[P])
        if li + 1 < len(params):
            x_parts = (side1, rest)
            b_prev = b.reshape(1, fout)
            s_prev = s_cur
        else:
            return _max_reduce(side1, rest, b, s_cur, n)


# speedup vs baseline: 3.6197x; 3.6197x over previous
import functools

import jax
import jax.numpy as jnp
from jax import lax
from jax.experimental import pallas as pl
from jax.experimental.pallas import tpu as pltpu
from jax.experimental.pallas import tpu_sc as plsc

NC = 2
NS = 16
CHUNK = 128
ZROWS = 64
BLK = 4000


def _elu(v):
    return jnp.where(v > 0, v, jnp.exp(jnp.minimum(v, 0.0)) - 1.0)


def _pad16(s):
    return (s + 15) // 16 * 16


@functools.cache
def _make_seg_sum(n_nodes, e_pad, P, R, passes):
    RA = R + 128
    assert R % 128 == 0
    ra_per = RA // NS
    z_full, z_rem = divmod(ra_per, ZROWS)
    r_per = R // NS
    out_rows = passes * NC * R
    int_min = jnp.int32(-2147483648)

    mesh = plsc.VectorSubcoreMesh(
        core_axis_name="c", subcore_axis_name="s",
        num_cores=NC, num_subcores=NS)

    @functools.partial(
        pl.kernel,
        out_type=jax.ShapeDtypeStruct((out_rows, P), jnp.float32),
        mesh=mesh,
        scratch_types=[
            pltpu.VMEM((CHUNK,), jnp.int32),
            pltpu.VMEM((CHUNK,), jnp.int32),
            pltpu.VMEM((CHUNK, P), jnp.float32),
            pltpu.VMEM((ZROWS, P), jnp.float32),
            pltpu.VMEM((16,), jnp.int32),
            pltpu.VMEM_SHARED((RA, P), jnp.float32),
            pltpu.SemaphoreType.DMA,
        ],
        compiler_params=pltpu.CompilerParams(use_tc_tiling_on_sc=False),
    )
    def seg(vals_hbm, ssrc_hbm, sdst_hbm, bounds_hbm, out_hbm,
            src_v, dst_v, rows_v, zero_v, bnd_v, acc_sh, sem):
        c = lax.axis_index("c")
        s = lax.axis_index("s")
        lanes = lax.iota(jnp.int32, 16)
        zvec = jnp.zeros((16,), jnp.float32)
        for r in range(ZROWS):
            for j in range(P // 16):
                zero_v[r, pl.ds(j * 16, 16)] = zvec

        for p in range(passes):
            zbase = s * ra_per
            def zero_body(i, _):
                pltpu.sync_copy(zero_v, acc_sh.at[pl.ds(zbase + i * ZROWS, ZROWS)])
                return 0
            lax.fori_loop(0, z_full, zero_body, 0)
            if z_rem:
                pltpu.sync_copy(zero_v.at[pl.ds(0, z_rem)],
                                acc_sh.at[pl.ds(zbase + z_full * ZROWS, z_rem)])
            plsc.subcore_barrier()

            pltpu.sync_copy(bounds_hbm.at[p, c], bnd_v)
            bv = bnd_v[...]
            e_lo = bv[0]
            e_hi = bv[1]
            node_lo = (p * NC + c) * R
            nchunks = lax.shift_right_arithmetic(e_hi - e_lo + (CHUNK - 1), 7)
            nloc = lax.shift_right_arithmetic(nchunks - s + (NS - 1), 4)

            def chunk_body(i, _):
                base = pl.multiple_of(e_lo + (s + i * NS) * CHUNK, 8)
                pltpu.sync_copy(ssrc_hbm.at[pl.ds(base, CHUNK)], src_v)
                pltpu.sync_copy(sdst_hbm.at[pl.ds(base, CHUNK)], dst_v)
                for j in range(CHUNK // 16):
                    d = dst_v[pl.ds(j * 16, 16)]
                    loc = d - node_lo
                    ok = (loc >= 0) & (loc < R)
                    dst_v[pl.ds(j * 16, 16)] = jnp.where(ok, loc, R)
                pltpu.async_copy(vals_hbm.at[src_v], rows_v, sem).wait()
                pltpu.sync_copy(rows_v, acc_sh.at[dst_v], add=True)
                return 0
            lax.fori_loop(0, nloc, chunk_body, 0)
            plsc.subcore_barrier()

            out_base = node_lo + s * r_per
            pltpu.sync_copy(acc_sh.at[pl.ds(s * r_per, r_per)],
                            out_hbm.at[pl.ds(out_base, r_per)])
            if p + 1 < passes:
                plsc.subcore_barrier()

    return seg


def _mm_layer(x_parts, b_prev, s_prev, invn, W, s_cur):
    n = invn.shape[0]
    fin, fout = W.shape
    P_cur = _pad16(s_cur)
    grid = (n // BLK,)

    if b_prev is None:
        (pos,) = x_parts

        def body(x_ref, invn_ref, w_ref, norm_ref, rest_ref):
            sup = jnp.dot(x_ref[...], w_ref[...],
                          preferred_element_type=jnp.float32)
            scaled = sup[:, :s_cur] * invn_ref[...]
            norm_ref[...] = jnp.concatenate(
                [scaled, jnp.zeros((BLK, P_cur - s_cur), jnp.float32)], axis=1)
            rest_ref[...] = sup[:, s_cur:]

        in_specs = [
            pl.BlockSpec((BLK, pos.shape[1]), lambda i: (i, 0)),
            pl.BlockSpec((BLK, 1), lambda i: (i, 0)),
            pl.BlockSpec((fin, fout), lambda i: (0, 0)),
        ]
        args = (pos, invn, W)
    else:
        side1, rest_in = x_parts
        r_prev = rest_in.shape[1]

        def body(s1_ref, rest_ref_in, b_ref, invn_ref, w_ref,
                 norm_ref, rest_ref):
            x = jnp.concatenate([s1_ref[:, :s_prev], rest_ref_in[...]],
                                axis=1) + b_ref[...]
            x = _elu(x)
            sup = jnp.dot(x, w_ref[...], preferred_element_type=jnp.float32)
            scaled = sup[:, :s_cur] * invn_ref[...]
            norm_ref[...] = jnp.concatenate(
                [scaled, jnp.zeros((BLK, P_cur - s_cur), jnp.float32)], axis=1)
            rest_ref[...] = sup[:, s_cur:]

        in_specs = [
            pl.BlockSpec((BLK, side1.shape[1]), lambda i: (i, 0)),
            pl.BlockSpec((BLK, r_prev), lambda i: (i, 0)),
            pl.BlockSpec((1, fin), lambda i: (0, 0)),
            pl.BlockSpec((BLK, 1), lambda i: (i, 0)),
            pl.BlockSpec((fin, fout), lambda i: (0, 0)),
        ]
        args = (side1, rest_in, b_prev, invn, W)

    return pl.pallas_call(
        body,
        grid=grid,
        in_specs=in_specs,
        out_specs=[
            pl.BlockSpec((BLK, P_cur), lambda i: (i, 0)),
            pl.BlockSpec((BLK, fout - s_cur), lambda i: (i, 0)),
        ],
        out_shape=[
            jax.ShapeDtypeStruct((n, P_cur), jnp.float32),
            jax.ShapeDtypeStruct((n, fout - s_cur), jnp.float32),
        ],
    )(*args)


def _max_reduce(side1, rest, b, s_prev, n):
    fout = b.shape[0]

    def body(s1_ref, rest_ref, b_ref, out_ref):
        i = pl.program_id(0)
        blk = jnp.concatenate([s1_ref[:, :s_prev], rest_ref[...]], axis=1)
        m = jnp.max(blk, axis=0, keepdims=True)

        @pl.when(i == 0)
        def _():
            out_ref[...] = m

        @pl.when(i > 0)
        def _():
            out_ref[...] = jnp.maximum(out_ref[...], m)

        @pl.when(i == pl.num_programs(0) - 1)
        def _():
            out_ref[...] = _elu(out_ref[...] + b_ref[...])

    out = pl.pallas_call(
        body,
        grid=(n // BLK,),
        in_specs=[
            pl.BlockSpec((BLK, side1.shape[1]), lambda i: (i, 0)),
            pl.BlockSpec((BLK, rest.shape[1]), lambda i: (i, 0)),
            pl.BlockSpec((1, fout), lambda i: (0, 0)),
        ],
        out_specs=pl.BlockSpec((1, fout), lambda i: (0, 0)),
        out_shape=jax.ShapeDtypeStruct((1, fout), jnp.float32),
    )(side1, rest, b.reshape(1, fout))
    return out.reshape(fout)


def _seg_plan(P):
    if P <= 64:
        return 25088, 2
    return 12800, 4


def kernel(positions, adj, params):
    n = positions.shape[0]
    e = adj.shape[1]
    src = adj[0].astype(jnp.int32)
    dst = adj[1].astype(jnp.int32)

    sdst, ssrc = lax.sort_key_val(dst, src)
    rp = jnp.searchsorted(
        sdst, jnp.arange(n + 1, dtype=jnp.int32), side="left",
        method="sort").astype(jnp.int32)
    deg = (rp[1:] - rp[:-1]).astype(jnp.float32)
    invn = (1.0 / jnp.maximum(deg, 1.0))[:, None]
    ssrc_pad = jnp.concatenate([ssrc, jnp.zeros((CHUNK,), jnp.int32)])
    sdst_pad = jnp.concatenate([sdst, jnp.full((CHUNK,), n, jnp.int32)])
    e_pad = e + CHUNK

    def mk_bounds(R, passes):
        units = jnp.arange(passes * NC, dtype=jnp.int32) * R
        lo = jnp.minimum(units, n)
        hi = jnp.minimum(units + R, n)
        a = (rp[lo] // 8) * 8
        e_hi = rp[hi]
        bounds = jnp.zeros((passes * NC, 16), jnp.int32)
        bounds = bounds.at[:, 0].set(a).at[:, 1].set(e_hi)
        return bounds.reshape(passes, NC, 16)

    plans = {}
    for (_, fo) in [(None, W.shape[1]) for (W, _) in params]:
        P = _pad16(max(fo // 3, 2))
        if P not in plans:
            plans[P] = _seg_plan(P)
    bounds = {P: mk_bounds(R, passes) for P, (R, passes) in plans.items()}

    x_parts = (positions,)
    b_prev = None
    s_prev = None
    for li, (W, b) in enumerate(params):
        fout = W.shape[1]
        s_cur = max(fout // 3, 2)
        P = _pad16(s_cur)
        R, passes = plans[P]
        normed, rest = _mm_layer(x_parts, b_prev, s_prev, invn, W, s_cur)
        seg = _make_seg_sum(n, e_pad, P, R, passes)
        side1 = seg(normed, ssrc_pad, sdst_pad, bounds[P])
        if li + 1 < len(params):
            x_parts = (side1, rest)
            b_prev = b.reshape(1, fout)
            s_prev = s_cur
        else:
            return _max_reduce(side1, rest, b, s_cur, n)


# speedup vs baseline: 6.3508x; 1.7545x over previous
import functools

import jax
import jax.numpy as jnp
from jax import lax
from jax.experimental import pallas as pl
from jax.experimental.pallas import tpu as pltpu
from jax.experimental.pallas import tpu_sc as plsc

NC = 2
NS = 16
CHUNK = 128
ZROWS = 64
BLK = 4000


def _elu(v):
    return jnp.where(v > 0, v, jnp.exp(jnp.minimum(v, 0.0)) - 1.0)


def _pad16(s):
    return (s + 15) // 16 * 16


@functools.cache
def _make_seg_sum(n_nodes, e_pad, P, R, passes):
    RA = R + 128
    assert R % 128 == 0
    ra_per = RA // NS
    z_full, z_rem = divmod(ra_per, ZROWS)
    r_per = R // NS
    out_rows = passes * NC * R

    mesh = plsc.VectorSubcoreMesh(
        core_axis_name="c", subcore_axis_name="s",
        num_cores=NC, num_subcores=NS)

    @functools.partial(
        pl.kernel,
        out_type=jax.ShapeDtypeStruct((out_rows, P), jnp.float32),
        mesh=mesh,
        scratch_types=[
            pltpu.VMEM((CHUNK,), jnp.int32),
            pltpu.VMEM((CHUNK,), jnp.int32),
            pltpu.VMEM((CHUNK, P), jnp.float32),
            pltpu.VMEM((ZROWS, P), jnp.float32),
            pltpu.VMEM((16,), jnp.int32),
            pltpu.VMEM_SHARED((RA, P), jnp.float32),
            pltpu.SemaphoreType.DMA,
        ],
        compiler_params=pltpu.CompilerParams(use_tc_tiling_on_sc=False),
    )
    def seg(vals_hbm, ssrc_hbm, sdst_hbm, bounds_hbm, out_hbm,
            src_v, dst_v, rows_v, zero_v, bnd_v, acc_sh, sem):
        c = lax.axis_index("c")
        s = lax.axis_index("s")
        zvec = jnp.zeros((16,), jnp.float32)
        for r in range(ZROWS):
            for j in range(P // 16):
                zero_v[r, pl.ds(j * 16, 16)] = zvec

        for p in range(passes):
            zbase = s * ra_per
            def zero_body(i, _):
                pltpu.sync_copy(zero_v, acc_sh.at[pl.ds(zbase + i * ZROWS, ZROWS)])
                return 0
            lax.fori_loop(0, z_full, zero_body, 0)
            if z_rem:
                pltpu.sync_copy(zero_v.at[pl.ds(0, z_rem)],
                                acc_sh.at[pl.ds(zbase + z_full * ZROWS, z_rem)])
            plsc.subcore_barrier()

            pltpu.sync_copy(bounds_hbm.at[p, c], bnd_v)
            bv = bnd_v[...]
            e_lo = bv[0]
            e_hi = bv[1]
            node_lo = (p * NC + c) * R
            nchunks = lax.shift_right_arithmetic(e_hi - e_lo + (CHUNK - 1), 7)
            nloc = lax.shift_right_arithmetic(nchunks - s + (NS - 1), 4)

            def chunk_body(i, _):
                base = pl.multiple_of(e_lo + (s + i * NS) * CHUNK, 8)
                pltpu.sync_copy(ssrc_hbm.at[pl.ds(base, CHUNK)], src_v)
                pltpu.sync_copy(sdst_hbm.at[pl.ds(base, CHUNK)], dst_v)
                for j in range(CHUNK // 16):
                    d = dst_v[pl.ds(j * 16, 16)]
                    loc = d - node_lo
                    ok = (loc >= 0) & (loc < R)
                    dst_v[pl.ds(j * 16, 16)] = jnp.where(ok, loc, R)
                pltpu.async_copy(vals_hbm.at[src_v], rows_v, sem).wait()
                pltpu.sync_copy(rows_v, acc_sh.at[dst_v], add=True)
                return 0
            lax.fori_loop(0, nloc, chunk_body, 0)
            plsc.subcore_barrier()

            out_base = node_lo + s * r_per
            pltpu.sync_copy(acc_sh.at[pl.ds(s * r_per, r_per)],
                            out_hbm.at[pl.ds(out_base, r_per)])
            if p + 1 < passes:
                plsc.subcore_barrier()

    return seg


@functools.cache
def _make_deg(n_nodes, e_pad, R, passes):
    P = 16
    RA = R + 128
    ra_per = RA // NS
    z_full, z_rem = divmod(ra_per, ZROWS)
    r_per = R // NS
    out_rows = passes * NC * R

    mesh = plsc.VectorSubcoreMesh(
        core_axis_name="c", subcore_axis_name="s",
        num_cores=NC, num_subcores=NS)

    @functools.partial(
        pl.kernel,
        out_type=jax.ShapeDtypeStruct((out_rows, P), jnp.float32),
        mesh=mesh,
        scratch_types=[
            pltpu.VMEM((CHUNK,), jnp.int32),
            pltpu.VMEM((CHUNK, P), jnp.float32),
            pltpu.VMEM((ZROWS, P), jnp.float32),
            pltpu.VMEM((16,), jnp.int32),
            pltpu.VMEM_SHARED((RA, P), jnp.float32),
        ],
        compiler_params=pltpu.CompilerParams(use_tc_tiling_on_sc=False),
    )
    def degk(sdst_hbm, bounds_hbm, out_hbm, dst_v, ones_v, zero_v, bnd_v, acc_sh):
        c = lax.axis_index("c")
        s = lax.axis_index("s")
        zvec = jnp.zeros((16,), jnp.float32)
        ovec = jnp.full((16,), 1.0, jnp.float32)
        for r in range(ZROWS):
            zero_v[r, pl.ds(0, 16)] = zvec
        for r in range(CHUNK):
            ones_v[r, pl.ds(0, 16)] = ovec

        for p in range(passes):
            zbase = s * ra_per
            def zero_body(i, _):
                pltpu.sync_copy(zero_v, acc_sh.at[pl.ds(zbase + i * ZROWS, ZROWS)])
                return 0
            lax.fori_loop(0, z_full, zero_body, 0)
            if z_rem:
                pltpu.sync_copy(zero_v.at[pl.ds(0, z_rem)],
                                acc_sh.at[pl.ds(zbase + z_full * ZROWS, z_rem)])
            plsc.subcore_barrier()

            pltpu.sync_copy(bounds_hbm.at[p, c], bnd_v)
            bv = bnd_v[...]
            e_lo = bv[0]
            e_hi = bv[1]
            node_lo = (p * NC + c) * R
            nchunks = lax.shift_right_arithmetic(e_hi - e_lo + (CHUNK - 1), 7)
            nloc = lax.shift_right_arithmetic(nchunks - s + (NS - 1), 4)

            def chunk_body(i, _):
                base = pl.multiple_of(e_lo + (s + i * NS) * CHUNK, 8)
                pltpu.sync_copy(sdst_hbm.at[pl.ds(base, CHUNK)], dst_v)
                for j in range(CHUNK // 16):
                    d = dst_v[pl.ds(j * 16, 16)]
                    loc = d - node_lo
                    ok = (loc >= 0) & (loc < R)
                    dst_v[pl.ds(j * 16, 16)] = jnp.where(ok, loc, R)
                pltpu.sync_copy(ones_v, acc_sh.at[dst_v], add=True)
                return 0
            lax.fori_loop(0, nloc, chunk_body, 0)
            plsc.subcore_barrier()

            out_base = node_lo + s * r_per
            pltpu.sync_copy(acc_sh.at[pl.ds(s * r_per, r_per)],
                            out_hbm.at[pl.ds(out_base, r_per)])
            if p + 1 < passes:
                plsc.subcore_barrier()

    return degk


def _mm_layer(x_parts, b_prev, s_prev, deg, W, s_cur, n):
    fin, fout = W.shape
    P_cur = _pad16(s_cur)
    grid = (n // BLK,)

    if b_prev is None:
        (pos,) = x_parts

        def body(x_ref, deg_ref, w_ref, norm_ref, rest_ref):
            sup = jnp.dot(x_ref[...], w_ref[...],
                          preferred_element_type=jnp.float32)
            invn = 1.0 / jnp.maximum(deg_ref[:, :1], 1.0)
            scaled = sup[:, :s_cur] * invn
            norm_ref[...] = jnp.concatenate(
                [scaled, jnp.zeros((BLK, P_cur - s_cur), jnp.float32)], axis=1)
            rest_ref[...] = sup[:, s_cur:]

        in_specs = [
            pl.BlockSpec((BLK, pos.shape[1]), lambda i: (i, 0)),
            pl.BlockSpec((BLK, 16), lambda i: (i, 0)),
            pl.BlockSpec((fin, fout), lambda i: (0, 0)),
        ]
        args = (pos, deg, W)
    else:
        side1, rest_in = x_parts
        r_prev = rest_in.shape[1]

        def body(s1_ref, rest_ref_in, b_ref, deg_ref, w_ref,
                 norm_ref, rest_ref):
            x = jnp.concatenate([s1_ref[:, :s_prev], rest_ref_in[...]],
                                axis=1) + b_ref[...]
            x = _elu(x)
            sup = jnp.dot(x, w_ref[...], preferred_element_type=jnp.float32)
            invn = 1.0 / jnp.maximum(deg_ref[:, :1], 1.0)
            scaled = sup[:, :s_cur] * invn
            norm_ref[...] = jnp.concatenate(
                [scaled, jnp.zeros((BLK, P_cur - s_cur), jnp.float32)], axis=1)
            rest_ref[...] = sup[:, s_cur:]

        in_specs = [
            pl.BlockSpec((BLK, side1.shape[1]), lambda i: (i, 0)),
            pl.BlockSpec((BLK, r_prev), lambda i: (i, 0)),
            pl.BlockSpec((1, fin), lambda i: (0, 0)),
            pl.BlockSpec((BLK, 16), lambda i: (i, 0)),
            pl.BlockSpec((fin, fout), lambda i: (0, 0)),
        ]
        args = (side1, rest_in, b_prev, deg, W)

    return pl.pallas_call(
        body,
        grid=grid,
        in_specs=in_specs,
        out_specs=[
            pl.BlockSpec((BLK, P_cur), lambda i: (i, 0)),
            pl.BlockSpec((BLK, fout - s_cur), lambda i: (i, 0)),
        ],
        out_shape=[
            jax.ShapeDtypeStruct((n, P_cur), jnp.float32),
            jax.ShapeDtypeStruct((n, fout - s_cur), jnp.float32),
        ],
    )(*args)


def _max_reduce(side1, rest, b, s_prev, n):
    fout = b.shape[0]

    def body(s1_ref, rest_ref, b_ref, out_ref):
        i = pl.program_id(0)
        blk = jnp.concatenate([s1_ref[:, :s_prev], rest_ref[...]], axis=1)
        m = jnp.max(blk, axis=0, keepdims=True)

        @pl.when(i == 0)
        def _():
            out_ref[...] = m

        @pl.when(i > 0)
        def _():
            out_ref[...] = jnp.maximum(out_ref[...], m)

        @pl.when(i == pl.num_programs(0) - 1)
        def _():
            out_ref[...] = _elu(out_ref[...] + b_ref[...])

    out = pl.pallas_call(
        body,
        grid=(n // BLK,),
        in_specs=[
            pl.BlockSpec((BLK, side1.shape[1]), lambda i: (i, 0)),
            pl.BlockSpec((BLK, rest.shape[1]), lambda i: (i, 0)),
            pl.BlockSpec((1, fout), lambda i: (0, 0)),
        ],
        out_specs=pl.BlockSpec((1, fout), lambda i: (0, 0)),
        out_shape=jax.ShapeDtypeStruct((1, fout), jnp.float32),
    )(side1, rest, b.reshape(1, fout))
    return out.reshape(fout)


def _seg_plan(P):
    if P <= 64:
        return 25088, 2
    return 12800, 4


def kernel(positions, adj, params):
    n = positions.shape[0]
    e = adj.shape[1]
    src = adj[0].astype(jnp.int32)
    dst = adj[1].astype(jnp.int32)

    sdst, ssrc = lax.sort_key_val(dst, src)
    ssrc_pad = jnp.concatenate([ssrc, jnp.zeros((CHUNK,), jnp.int32)])
    sdst_pad = jnp.concatenate([sdst, jnp.full((CHUNK,), n, jnp.int32)])
    e_pad = e + CHUNK

    def mk_bounds(R, passes):
        units = jnp.minimum(jnp.arange(passes * NC + 1) * R, n)
        rp = jnp.searchsorted(sdst, units.astype(jnp.int32), side="left")
        a = (rp[:-1] // 8) * 8
        e_hi = rp[1:]
        bounds = jnp.zeros((passes * NC, 16), jnp.int32)
        bounds = bounds.at[:, 0].set(a).at[:, 1].set(e_hi)
        return bounds.reshape(passes, NC, 16)

    plans = {}
    for (W, _) in params:
        P = _pad16(max(W.shape[1] // 3, 2))
        if P not in plans:
            plans[P] = _seg_plan(P)
    bounds = {P: mk_bounds(R, passes) for P, (R, passes) in plans.items()}

    dR, dpasses = _seg_plan(16)
    deg = _make_deg(n, e_pad, dR, dpasses)(sdst_pad, bounds[48] if 48 in plans
                                           else mk_bounds(dR, dpasses))

    x_parts = (positions,)
    b_prev = None
    s_prev = None
    for li, (W, b) in enumerate(params):
        fout = W.shape[1]
        s_cur = max(fout // 3, 2)
        P = _pad16(s_cur)
        R, passes = plans[P]
        normed, rest = _mm_layer(x_parts, b_prev, s_prev, deg, W, s_cur, n)
        seg = _make_seg_sum(n, e_pad, P, R, passes)
        side1 = seg(normed, ssrc_pad, sdst_pad, bounds[P])
        if li + 1 < len(params):
            x_parts = (side1, rest)
            b_prev = b.reshape(1, fout)
            s_prev = s_cur
        else:
            return _max_reduce(side1, rest, b, s_cur, n)


# speedup vs baseline: 8.7170x; 1.3726x over previous
import functools

import jax
import jax.numpy as jnp
from jax import lax
from jax.experimental import pallas as pl
from jax.experimental.pallas import tpu as pltpu
from jax.experimental.pallas import tpu_sc as plsc

NC = 2
NS = 16
CHUNK = 128
ZROWS = 64
BLK = 4000


def _elu(v):
    return jnp.where(v > 0, v, jnp.exp(jnp.minimum(v, 0.0)) - 1.0)


def _pad16(s):
    return (s + 15) // 16 * 16


@functools.cache
def _make_seg_sum(n_nodes, e_pad, P, R, passes):
    RA = R + 128
    assert R % 128 == 0
    ra_per = RA // NS
    z_full, z_rem = divmod(ra_per, ZROWS)
    r_per = R // NS
    out_rows = passes * NC * R

    mesh = plsc.VectorSubcoreMesh(
        core_axis_name="c", subcore_axis_name="s",
        num_cores=NC, num_subcores=NS)

    @functools.partial(
        pl.kernel,
        out_type=jax.ShapeDtypeStruct((out_rows, P), jnp.float32),
        mesh=mesh,
        scratch_types=[
            pltpu.VMEM((CHUNK,), jnp.int32),
            pltpu.VMEM((CHUNK,), jnp.int32),
            pltpu.VMEM((CHUNK, P), jnp.float32),
            pltpu.VMEM((CHUNK,), jnp.int32),
            pltpu.VMEM((CHUNK,), jnp.int32),
            pltpu.VMEM((CHUNK, P), jnp.float32),
            pltpu.VMEM((ZROWS, P), jnp.float32),
            pltpu.VMEM((16,), jnp.int32),
            pltpu.VMEM_SHARED((RA, P), jnp.float32),
            pltpu.SemaphoreType.DMA,
            pltpu.SemaphoreType.DMA,
        ],
        compiler_params=pltpu.CompilerParams(use_tc_tiling_on_sc=False),
    )
    def seg(vals_hbm, ssrc_hbm, sdst_hbm, bounds_hbm, out_hbm,
            src_v, dst_v, rows_v, src_w, dst_w, rows_w, zero_v, bnd_v,
            acc_sh, sem_a, sem_b):
        c = lax.axis_index("c")
        s = lax.axis_index("s")
        zvec = jnp.zeros((16,), jnp.float32)
        for r in range(ZROWS):
            for j in range(P // 16):
                zero_v[r, pl.ds(j * 16, 16)] = zvec

        for p in range(passes):
            zbase = s * ra_per
            def zero_body(i, _):
                pltpu.sync_copy(zero_v, acc_sh.at[pl.ds(zbase + i * ZROWS, ZROWS)])
                return 0
            lax.fori_loop(0, z_full, zero_body, 0)
            if z_rem:
                pltpu.sync_copy(zero_v.at[pl.ds(0, z_rem)],
                                acc_sh.at[pl.ds(zbase + z_full * ZROWS, z_rem)])
            plsc.subcore_barrier()

            pltpu.sync_copy(bounds_hbm.at[p, c], bnd_v)
            bv = bnd_v[...]
            e_lo = bv[0]
            e_hi = bv[1]
            node_lo = (p * NC + c) * R
            nchunks = lax.shift_right_arithmetic(e_hi - e_lo + (CHUNK - 1), 7)
            nloc = lax.shift_right_arithmetic(nchunks - s + (NS - 1), 4)

            def fire(src_b, dst_b, rows_b, sem_x, i):
                base = pl.multiple_of(e_lo + (s + i * NS) * CHUNK, 8)
                pltpu.sync_copy(ssrc_hbm.at[pl.ds(base, CHUNK)], src_b)
                pltpu.sync_copy(sdst_hbm.at[pl.ds(base, CHUNK)], dst_b)
                for j in range(CHUNK // 16):
                    d = dst_b[pl.ds(j * 16, 16)]
                    loc = d - node_lo
                    ok = (loc >= 0) & (loc < R)
                    dst_b[pl.ds(j * 16, 16)] = jnp.where(ok, loc, R)
                pltpu.async_copy(vals_hbm.at[src_b], rows_b, sem_x)

            def drain(src_b, dst_b, rows_b, sem_x):
                pltpu.make_async_copy(vals_hbm.at[src_b], rows_b, sem_x).wait()
                pltpu.sync_copy(rows_b, acc_sh.at[dst_b], add=True)

            @pl.when(nloc > 0)
            def _():
                fire(src_v, dst_v, rows_v, sem_a, 0)

            def pair_body(k, _):
                i1 = 2 * k + 1
                i2 = 2 * k + 2

                @pl.when(i1 < nloc)
                def _():
                    fire(src_w, dst_w, rows_w, sem_b, i1)
                drain(src_v, dst_v, rows_v, sem_a)

                @pl.when(i2 < nloc)
                def _():
                    fire(src_v, dst_v, rows_v, sem_a, i2)

                @pl.when(i1 < nloc)
                def _():
                    drain(src_w, dst_w, rows_w, sem_b)
                return 0
            lax.fori_loop(0, (nloc + 1) >> 1, pair_body, 0)
            plsc.subcore_barrier()

            out_base = node_lo + s * r_per
            pltpu.sync_copy(acc_sh.at[pl.ds(s * r_per, r_per)],
                            out_hbm.at[pl.ds(out_base, r_per)])
            if p + 1 < passes:
                plsc.subcore_barrier()

    return seg


@functools.cache
def _make_deg(n_nodes, e_pad, R, passes):
    P = 16
    RA = R + 128
    ra_per = RA // NS
    z_full, z_rem = divmod(ra_per, ZROWS)
    r_per = R // NS
    out_rows = passes * NC * R

    mesh = plsc.VectorSubcoreMesh(
        core_axis_name="c", subcore_axis_name="s",
        num_cores=NC, num_subcores=NS)

    @functools.partial(
        pl.kernel,
        out_type=jax.ShapeDtypeStruct((out_rows, P), jnp.float32),
        mesh=mesh,
        scratch_types=[
            pltpu.VMEM((CHUNK,), jnp.int32),
            pltpu.VMEM((CHUNK, P), jnp.float32),
            pltpu.VMEM((ZROWS, P), jnp.float32),
            pltpu.VMEM((16,), jnp.int32),
            pltpu.VMEM_SHARED((RA, P), jnp.float32),
        ],
        compiler_params=pltpu.CompilerParams(use_tc_tiling_on_sc=False),
    )
    def degk(sdst_hbm, bounds_hbm, out_hbm, dst_v, ones_v, zero_v, bnd_v, acc_sh):
        c = lax.axis_index("c")
        s = lax.axis_index("s")
        zvec = jnp.zeros((16,), jnp.float32)
        ovec = jnp.full((16,), 1.0, jnp.float32)
        for r in range(ZROWS):
            zero_v[r, pl.ds(0, 16)] = zvec
        for r in range(CHUNK):
            ones_v[r, pl.ds(0, 16)] = ovec

        for p in range(passes):
            zbase = s * ra_per
            def zero_body(i, _):
                pltpu.sync_copy(zero_v, acc_sh.at[pl.ds(zbase + i * ZROWS, ZROWS)])
                return 0
            lax.fori_loop(0, z_full, zero_body, 0)
            if z_rem:
                pltpu.sync_copy(zero_v.at[pl.ds(0, z_rem)],
                                acc_sh.at[pl.ds(zbase + z_full * ZROWS, z_rem)])
            plsc.subcore_barrier()

            pltpu.sync_copy(bounds_hbm.at[p, c], bnd_v)
            bv = bnd_v[...]
            e_lo = bv[0]
            e_hi = bv[1]
            node_lo = (p * NC + c) * R
            nchunks = lax.shift_right_arithmetic(e_hi - e_lo + (CHUNK - 1), 7)
            nloc = lax.shift_right_arithmetic(nchunks - s + (NS - 1), 4)

            def chunk_body(i, _):
                base = pl.multiple_of(e_lo + (s + i * NS) * CHUNK, 8)
                pltpu.sync_copy(sdst_hbm.at[pl.ds(base, CHUNK)], dst_v)
                for j in range(CHUNK // 16):
                    d = dst_v[pl.ds(j * 16, 16)]
                    loc = d - node_lo
                    ok = (loc >= 0) & (loc < R)
                    dst_v[pl.ds(j * 16, 16)] = jnp.where(ok, loc, R)
                pltpu.sync_copy(ones_v, acc_sh.at[dst_v], add=True)
                return 0
            lax.fori_loop(0, nloc, chunk_body, 0)
            plsc.subcore_barrier()

            out_base = node_lo + s * r_per
            pltpu.sync_copy(acc_sh.at[pl.ds(s * r_per, r_per)],
                            out_hbm.at[pl.ds(out_base, r_per)])
            if p + 1 < passes:
                plsc.subcore_barrier()

    return degk


def _mm_layer(x_parts, b_prev, s_prev, deg, W, s_cur, n):
    fin, fout = W.shape
    P_cur = _pad16(s_cur)
    grid = (n // BLK,)

    if b_prev is None:
        (pos,) = x_parts

        def body(x_ref, deg_ref, w_ref, norm_ref, rest_ref):
            sup = jnp.dot(x_ref[...], w_ref[...],
                          preferred_element_type=jnp.float32)
            invn = 1.0 / jnp.maximum(deg_ref[:, :1], 1.0)
            scaled = sup[:, :s_cur] * invn
            norm_ref[...] = jnp.concatenate(
                [scaled, jnp.zeros((BLK, P_cur - s_cur), jnp.float32)], axis=1)
            rest_ref[...] = sup[:, s_cur:]

        in_specs = [
            pl.BlockSpec((BLK, pos.shape[1]), lambda i: (i, 0)),
            pl.BlockSpec((BLK, 16), lambda i: (i, 0)),
            pl.BlockSpec((fin, fout), lambda i: (0, 0)),
        ]
        args = (pos, deg, W)
    else:
        side1, rest_in = x_parts
        r_prev = rest_in.shape[1]

        def body(s1_ref, rest_ref_in, b_ref, deg_ref, w_ref,
                 norm_ref, rest_ref):
            x = jnp.concatenate([s1_ref[:, :s_prev], rest_ref_in[...]],
                                axis=1) + b_ref[...]
            x = _elu(x)
            sup = jnp.dot(x, w_ref[...], preferred_element_type=jnp.float32)
            invn = 1.0 / jnp.maximum(deg_ref[:, :1], 1.0)
            scaled = sup[:, :s_cur] * invn
            norm_ref[...] = jnp.concatenate(
                [scaled, jnp.zeros((BLK, P_cur - s_cur), jnp.float32)], axis=1)
            rest_ref[...] = sup[:, s_cur:]

        in_specs = [
            pl.BlockSpec((BLK, side1.shape[1]), lambda i: (i, 0)),
            pl.BlockSpec((BLK, r_prev), lambda i: (i, 0)),
            pl.BlockSpec((1, fin), lambda i: (0, 0)),
            pl.BlockSpec((BLK, 16), lambda i: (i, 0)),
            pl.BlockSpec((fin, fout), lambda i: (0, 0)),
        ]
        args = (side1, rest_in, b_prev, deg, W)

    return pl.pallas_call(
        body,
        grid=grid,
        in_specs=in_specs,
        out_specs=[
            pl.BlockSpec((BLK, P_cur), lambda i: (i, 0)),
            pl.BlockSpec((BLK, fout - s_cur), lambda i: (i, 0)),
        ],
        out_shape=[
            jax.ShapeDtypeStruct((n, P_cur), jnp.float32),
            jax.ShapeDtypeStruct((n, fout - s_cur), jnp.float32),
        ],
    )(*args)


def _max_reduce(side1, rest, b, s_prev, n):
    fout = b.shape[0]

    def body(s1_ref, rest_ref, b_ref, out_ref):
        i = pl.program_id(0)
        blk = jnp.concatenate([s1_ref[:, :s_prev], rest_ref[...]], axis=1)
        m = jnp.max(blk, axis=0, keepdims=True)

        @pl.when(i == 0)
        def _():
            out_ref[...] = m

        @pl.when(i > 0)
        def _():
            out_ref[...] = jnp.maximum(out_ref[...], m)

        @pl.when(i == pl.num_programs(0) - 1)
        def _():
            out_ref[...] = _elu(out_ref[...] + b_ref[...])

    out = pl.pallas_call(
        body,
        grid=(n // BLK,),
        in_specs=[
            pl.BlockSpec((BLK, side1.shape[1]), lambda i: (i, 0)),
            pl.BlockSpec((BLK, rest.shape[1]), lambda i: (i, 0)),
            pl.BlockSpec((1, fout), lambda i: (0, 0)),
        ],
        out_specs=pl.BlockSpec((1, fout), lambda i: (0, 0)),
        out_shape=jax.ShapeDtypeStruct((1, fout), jnp.float32),
    )(side1, rest, b.reshape(1, fout))
    return out.reshape(fout)


def _seg_plan(P):
    if P <= 64:
        return 25088, 2
    return 12800, 4


def kernel(positions, adj, params):
    n = positions.shape[0]
    e = adj.shape[1]
    src = adj[0].astype(jnp.int32)
    dst = adj[1].astype(jnp.int32)

    sdst, ssrc = lax.sort_key_val(dst, src)
    ssrc_pad = jnp.concatenate([ssrc, jnp.zeros((CHUNK,), jnp.int32)])
    sdst_pad = jnp.concatenate([sdst, jnp.full((CHUNK,), n, jnp.int32)])
    e_pad = e + CHUNK

    def mk_bounds(R, passes):
        units = jnp.minimum(jnp.arange(passes * NC + 1) * R, n)
        rp = jnp.searchsorted(sdst, units.astype(jnp.int32), side="left")
        a = (rp[:-1] // 8) * 8
        e_hi = rp[1:]
        bounds = jnp.zeros((passes * NC, 16), jnp.int32)
        bounds = bounds.at[:, 0].set(a).at[:, 1].set(e_hi)
        return bounds.reshape(passes, NC, 16)

    plans = {}
    for (W, _) in params:
        P = _pad16(max(W.shape[1] // 3, 2))
        if P not in plans:
            plans[P] = _seg_plan(P)
    bounds = {P: mk_bounds(R, passes) for P, (R, passes) in plans.items()}

    dR, dpasses = _seg_plan(16)
    deg = _make_deg(n, e_pad, dR, dpasses)(sdst_pad, bounds[48] if 48 in plans
                                           else mk_bounds(dR, dpasses))

    x_parts = (positions,)
    b_prev = None
    s_prev = None
    for li, (W, b) in enumerate(params):
        fout = W.shape[1]
        s_cur = max(fout // 3, 2)
        P = _pad16(s_cur)
        R, passes = plans[P]
        normed, rest = _mm_layer(x_parts, b_prev, s_prev, deg, W, s_cur, n)
        seg = _make_seg_sum(n, e_pad, P, R, passes)
        side1 = seg(normed, ssrc_pad, sdst_pad, bounds[P])
        if li + 1 < len(params):
            x_parts = (side1, rest)
            b_prev = b.reshape(1, fout)
            s_prev = s_cur
        else:
            return _max_reduce(side1, rest, b, s_cur, n)
